# baseline (speedup 1.0000x reference)
import functools
import jax
import jax.numpy as jnp
from jax import lax
from jax.experimental import pallas as pl
from jax.experimental.pallas import tpu as pltpu
from jax.experimental.pallas import tpu_sc as plsc

D_MODEL = 64
SCALE = 8.0
CHUNK = 100
GCHUNK = 104
NC, NS, L = 2, 16, 16
NW = NC * NS
NBUF = 2


@jax.jit
def kernel(x, lut):
    b0, b1 = x.shape
    n = b0 * b1
    vocab = lut.shape[0]
    assert b1 == 2 * CHUNK
    n_chunks = n // CHUNK
    chunks_per_w = n_chunks // NW
    assert n_chunks % NW == 0 and chunks_per_w % NBUF == 0
    idx = jnp.pad(
        x.reshape(n_chunks, CHUNK).astype(jnp.int32),
        ((0, 0), (0, 128 - CHUNK)),
    )

    mesh = plsc.VectorSubcoreMesh(core_axis_name="c", subcore_axis_name="s")

    @functools.partial(
        pl.kernel,
        out_type=jax.ShapeDtypeStruct((b0, b1, D_MODEL), jnp.float32),
        mesh=mesh,
        compiler_params=pltpu.CompilerParams(use_tc_tiling_on_sc=False),
        scratch_types=[
            pltpu.VMEM((chunks_per_w, 128), jnp.int32),
            pltpu.VMEM((NBUF, GCHUNK, D_MODEL), jnp.float32),
            pltpu.VMEM((NBUF, CHUNK, D_MODEL), jnp.float32),
            pltpu.SemaphoreType.DMA((NBUF,)),
            pltpu.SemaphoreType.DMA((NBUF,)),
        ],
    )
    def run(lut_hbm, idx_hbm, out_hbm, idx_v, rows_v, obuf_v, gsem, wsem):
        wid = lax.axis_index("c") * NS + lax.axis_index("s")
        base = wid * chunks_per_w
        pltpu.sync_copy(idx_hbm.at[pl.ds(base, chunks_per_w)], idx_v)

        def gather(j, b):
            pltpu.make_async_copy(
                lut_hbm.at[idx_v.at[j, pl.ds(0, GCHUNK)]],
                rows_v.at[b],
                gsem.at[b],
            ).start()

        def wait_gather(j, b):
            pltpu.make_async_copy(
                lut_hbm.at[idx_v.at[j, pl.ds(0, GCHUNK)]],
                rows_v.at[b],
                gsem.at[b],
            ).wait()

        def out_block(g, b):
            return out_hbm.at[(base + g) // 2, pl.ds(b * CHUNK, CHUNK)]

        def writeback(g, b):
            pltpu.make_async_copy(
                obuf_v.at[b], out_block(g, b), wsem.at[b]
            ).start()

        def wait_writeback(g, b):
            pltpu.make_async_copy(
                obuf_v.at[b], out_block(g, b), wsem.at[b]
            ).wait()

        for b in range(NBUF):
            gather(b, b)

        @pl.loop(0, chunks_per_w, step=NBUF)
        def _ring(g):
            for b in range(NBUF):
                j = g + b
                wait_gather(j, b)

                @pl.when(j >= NBUF)
                def _reuse():
                    wait_writeback(g - NBUF, b)

                @pl.loop(0, CHUNK)
                def _row(r):
                    for c in range(D_MODEL // L):
                        sl = pl.ds(c * L, L)
                        obuf_v.at[b, r, sl][...] = (
                            rows_v.at[b, r, sl][...] * SCALE
                        )

                @pl.when(j + NBUF < chunks_per_w)
                def _refill():
                    gather(j + NBUF, b)

                writeback(g, b)

        for b in range(NBUF):
            wait_writeback(chunks_per_w - NBUF, b)

    return run(lut, idx)


# speedup vs baseline: 1.6390x; 1.6390x over previous
import functools
import jax
import jax.numpy as jnp
from jax import lax
from jax.experimental import pallas as pl
from jax.experimental.pallas import tpu as pltpu
from jax.experimental.pallas import tpu_sc as plsc

D_MODEL = 64
SCALE = 8.0
CHUNK = 100
GCHUNK = 104
NC, NS, L = 2, 16, 16
NW = NC * NS
NBUF = 2


@jax.jit
def kernel(x, lut):
    b0, b1 = x.shape
    n = b0 * b1
    vocab = lut.shape[0]
    assert b1 == 2 * CHUNK
    n_chunks = n // CHUNK
    chunks_per_w = n_chunks // NW
    assert n_chunks % NW == 0 and chunks_per_w % NBUF == 0
    pad_w = 128 - CHUNK
    pad_vals = (
        jnp.arange(pad_w, dtype=jnp.int32)[None, :]
        + jnp.arange(n_chunks, dtype=jnp.int32)[:, None] * 131
    ) % vocab
    idx = jnp.concatenate(
        [x.reshape(n_chunks, CHUNK).astype(jnp.int32), pad_vals], axis=1
    )

    mesh = plsc.VectorSubcoreMesh(core_axis_name="c", subcore_axis_name="s")

    @functools.partial(
        pl.kernel,
        out_type=jax.ShapeDtypeStruct((b0, b1, D_MODEL), jnp.float32),
        mesh=mesh,
        compiler_params=pltpu.CompilerParams(use_tc_tiling_on_sc=False),
        scratch_types=[
            pltpu.VMEM((chunks_per_w, 128), jnp.int32),
            pltpu.VMEM((NBUF, GCHUNK, D_MODEL), jnp.float32),
            pltpu.VMEM((NBUF, CHUNK, D_MODEL), jnp.float32),
            pltpu.SemaphoreType.DMA((NBUF,)),
            pltpu.SemaphoreType.DMA((NBUF,)),
        ],
    )
    def run(lut_hbm, idx_hbm, out_hbm, idx_v, rows_v, obuf_v, gsem, wsem):
        wid = lax.axis_index("c") * NS + lax.axis_index("s")
        base = wid * chunks_per_w
        pltpu.sync_copy(idx_hbm.at[pl.ds(base, chunks_per_w)], idx_v)

        def gather(j, b):
            pltpu.make_async_copy(
                lut_hbm.at[idx_v.at[j, pl.ds(0, GCHUNK)]],
                rows_v.at[b],
                gsem.at[b],
            ).start()

        def wait_gather(j, b):
            pltpu.make_async_copy(
                lut_hbm.at[idx_v.at[j, pl.ds(0, GCHUNK)]],
                rows_v.at[b],
                gsem.at[b],
            ).wait()

        def out_block(g, b):
            return out_hbm.at[(base + g) // 2, pl.ds(b * CHUNK, CHUNK)]

        def writeback(g, b):
            pltpu.make_async_copy(
                obuf_v.at[b], out_block(g, b), wsem.at[b]
            ).start()

        def wait_writeback(g, b):
            pltpu.make_async_copy(
                obuf_v.at[b], out_block(g, b), wsem.at[b]
            ).wait()

        for b in range(NBUF):
            gather(b, b)

        @pl.loop(0, chunks_per_w, step=NBUF)
        def _ring(g):
            for b in range(NBUF):
                j = g + b
                wait_gather(j, b)

                @pl.when(j >= NBUF)
                def _reuse():
                    wait_writeback(g - NBUF, b)

                @pl.loop(0, CHUNK)
                def _row(r):
                    for c in range(D_MODEL // L):
                        sl = pl.ds(c * L, L)
                        obuf_v.at[b, r, sl][...] = (
                            rows_v.at[b, r, sl][...] * SCALE
                        )

                @pl.when(j + NBUF < chunks_per_w)
                def _refill():
                    gather(j + NBUF, b)

                writeback(g, b)

        for b in range(NBUF):
            wait_writeback(chunks_per_w - NBUF, b)

    return run(lut, idx)


# speedup vs baseline: 2.0930x; 1.2770x over previous
import functools
import jax
import jax.numpy as jnp
from jax import lax
from jax.experimental import pallas as pl
from jax.experimental.pallas import tpu as pltpu
from jax.experimental.pallas import tpu_sc as plsc

D_MODEL = 64
SCALE = 8.0
CHUNK = 128
NC, NS, L = 2, 16, 16
NW = NC * NS
NBUF = 2


@jax.jit
def kernel(x, lut):
    b0, b1 = x.shape
    n = b0 * b1
    assert n % (NW * CHUNK) == 0
    n_chunks = n // CHUNK
    chunks_per_w = n_chunks // NW
    assert chunks_per_w % NBUF == 0
    idx = x.reshape(n_chunks, CHUNK).astype(jnp.int32)
    lutp = jnp.pad(lut, ((0, 0), (0, 2 * D_MODEL - lut.shape[1])))

    mesh = plsc.VectorSubcoreMesh(core_axis_name="c", subcore_axis_name="s")

    @functools.partial(
        pl.kernel,
        out_type=jax.ShapeDtypeStruct((n, D_MODEL), jnp.float32),
        mesh=mesh,
        compiler_params=pltpu.CompilerParams(use_tc_tiling_on_sc=True),
        scratch_types=[
            pltpu.VMEM((chunks_per_w, CHUNK), jnp.int32),
            pltpu.VMEM((NBUF, CHUNK, 2 * D_MODEL), jnp.float32),
            pltpu.VMEM((NBUF, CHUNK, D_MODEL), jnp.float32),
            pltpu.SemaphoreType.DMA((NBUF,)),
            pltpu.SemaphoreType.DMA((NBUF,)),
        ],
    )
    def run(lut_hbm, idx_hbm, out_hbm, idx_v, rows_v, obuf_v, gsem, wsem):
        wid = lax.axis_index("c") * NS + lax.axis_index("s")
        base = wid * chunks_per_w
        pltpu.sync_copy(idx_hbm.at[pl.ds(base, chunks_per_w)], idx_v)

        def gather(j, b):
            pltpu.make_async_copy(
                lut_hbm.at[idx_v.at[j]], rows_v.at[b], gsem.at[b]
            ).start()

        def wait_gather(j, b):
            pltpu.make_async_copy(
                lut_hbm.at[idx_v.at[j]], rows_v.at[b], gsem.at[b]
            ).wait()

        def out_block(j):
            return out_hbm.at[pl.ds((base + j) * CHUNK, CHUNK)]

        def writeback(j, b):
            pltpu.make_async_copy(
                obuf_v.at[b], out_block(j), wsem.at[b]
            ).start()

        def wait_writeback(j, b):
            pltpu.make_async_copy(
                obuf_v.at[b], out_block(j), wsem.at[b]
            ).wait()

        for b in range(NBUF):
            gather(b, b)

        @pl.loop(0, chunks_per_w, step=NBUF)
        def _ring(g):
            for b in range(NBUF):
                j = g + b
                wait_gather(j, b)

                @pl.when(j >= NBUF)
                def _reuse():
                    wait_writeback(j - NBUF, b)

                @pl.loop(0, CHUNK)
                def _row(r):
                    for c in range(D_MODEL // L):
                        sl = pl.ds(c * L, L)
                        obuf_v.at[b, r, sl][...] = (
                            rows_v.at[b, r, sl][...] * SCALE
                        )

                @pl.when(j + NBUF < chunks_per_w)
                def _refill():
                    gather(j + NBUF, b)

                writeback(j, b)

        for b in range(NBUF):
            wait_writeback(chunks_per_w - NBUF + b, b)

    out = run(lutp, idx)
    return out.reshape(b0, b1, D_MODEL)
